# baseline (speedup 1.0000x reference)
import functools

import jax
import jax.numpy as jnp
from jax import lax
from jax.experimental import pallas as pl
from jax.experimental.pallas import tpu as pltpu
from jax.experimental.pallas import tpu_sc as plsc

SEQ = 50
BATCH = 4096
EMBED = 32
LANES = 16
NC = 2
NS = 16
NW = NC * NS
BPW = BATCH // NW
GROUPS = BPW // LANES
NBUF = 2
VOCAB = 1000000
PACK = 4
PROWS = VOCAB // PACK
WPAD = SEQ * EMBED + LANES

TB = 512
TOUT = TB // PACK
NBLK = VOCAB // TB
FULL_ROUNDS = 61


def _mesh_kernel(body, out_shape, scratch):
    mesh = plsc.VectorSubcoreMesh(core_axis_name="c", subcore_axis_name="s")
    return functools.partial(
        pl.kernel,
        out_type=out_shape,
        mesh=mesh,
        compiler_params=pltpu.CompilerParams(
            needs_layout_passes=False, use_tc_tiling_on_sc=True),
        scratch_types=scratch,
    )(body)


RUNROLL = 8


def _tr_block(in_ref, out_ref, iota, n=None):
    if n is None:
        n = out_ref.shape[0]
    iota_hi = iota + LANES

    def rows(r8, _):
        r0 = r8 * RUNROLL
        for rr in range(RUNROLL):
            r = r0 + rr
            rsplat = jnp.broadcast_to(r, (LANES,))
            c4 = r * 4
            for h in range(4):
                col = jnp.broadcast_to(c4 + h, (LANES,))
                lo = plsc.load_gather(in_ref, [iota, col])
                hi = plsc.load_gather(in_ref, [iota_hi, col])
                plsc.store_scatter(out_ref, [rsplat, iota + (h * EMBED)], lo)
                plsc.store_scatter(
                    out_ref, [rsplat, iota + (h * EMBED + LANES)], hi)
        return 0

    lax.fori_loop(0, n // RUNROLL, rows, 0)


def _tr_body(tT_hbm, tail_hbm, t2_hbm, in_v, out_v, si0, si1, so0, so1):
    wid = lax.axis_index("s") * NC + lax.axis_index("c")
    iota = lax.iota(jnp.int32, LANES)
    sin = [si0, si1]
    sout = [so0, so1]

    def in_copy(k, buf):
        b = wid + k * NW
        off = pl.multiple_of(b * TB, 128)
        return pltpu.make_async_copy(
            tT_hbm.at[:, pl.ds(off, TB)], in_v.at[buf], sin[buf])

    def out_copy(k, buf):
        b = wid + k * NW
        off = pl.multiple_of(b * TOUT, 8)
        return pltpu.make_async_copy(
            out_v.at[buf], t2_hbm.at[pl.ds(off, TOUT), :], sout[buf])

    in_copy(0, 0).start()
    in_copy(1, 1).start()

    def round_(k, buf):
        in_copy(k, buf).wait()

        @pl.when(k >= NBUF)
        def _():
            out_copy(k - NBUF, buf).wait()
        _tr_block(in_v.at[buf], out_v.at[buf], iota)

        @pl.when(k + NBUF < FULL_ROUNDS)
        def _():
            in_copy(k + NBUF, buf).start()
        out_copy(k, buf).start()

    def outer(k2, carry):
        for half in range(NBUF):
            round_(k2 * NBUF + half, half)
        return carry

    lax.fori_loop(0, FULL_ROUNDS // NBUF, outer, 0)
    round_(FULL_ROUNDS - 1, 0)
    out_copy(FULL_ROUNDS - 2, 1).wait()
    out_copy(FULL_ROUNDS - 1, 0).wait()

    @pl.when(wid == 0)
    def _():
        b = NBLK - 1
        pltpu.sync_copy(tT_hbm.at[:, pl.ds(b * TB, TB)], in_v.at[0])
        _tr_block(in_v.at[0], out_v.at[0], iota)
        pltpu.sync_copy(out_v.at[0], t2_hbm.at[pl.ds(b * TOUT, TOUT), :])

    @pl.when(wid == 1)
    def _():
        pltpu.sync_copy(tail_hbm, out_v.at[1].at[pl.ds(0, 16)])
        pltpu.sync_copy(out_v.at[1].at[pl.ds(0, 16)],
                        t2_hbm.at[pl.ds(PROWS - 16, 16), :])


def _body(x_hbm, w_hbm, t2_hbm, out_hbm, idx_v, row_v, rows_v, w_v, out_v,
          sem0, sem1, semi):
    wid = lax.axis_index("s") * NC + lax.axis_index("c")
    base = wid * BPW

    for s in range(SEQ):
        pltpu.async_copy(x_hbm.at[pl.ds(s * BATCH + base, BPW)],
                         idx_v.at[s], semi)
    pltpu.sync_copy(w_hbm, w_v)
    for s in range(SEQ):
        pltpu.make_async_copy(x_hbm.at[pl.ds(s * BATCH + base, BPW)],
                              idx_v.at[s], semi).wait()

    for s in range(SEQ):
        for g in range(GROUPS):
            iv = idx_v[s, pl.ds(g * LANES, LANES)]
            row_v[s, pl.ds(g * LANES, LANES)] = lax.shift_right_logical(
                iv, 2)

    sems = [sem0, sem1]
    for buf in range(NBUF):
        pltpu.async_copy(t2_hbm.at[row_v.at[buf]], rows_v.at[buf],
                         sems[buf])

    iota = lax.iota(jnp.int32, LANES)
    three = jnp.full((LANES,), 3, jnp.int32)

    def step(s, buf, accs):
        pltpu.make_async_copy(t2_hbm.at[row_v.at[s]], rows_v.at[buf],
                              sems[buf]).wait()
        rows = rows_v.at[buf]
        out = list(accs)
        wbase = jnp.broadcast_to(s * EMBED, (LANES,))
        cbase = []
        for g in range(GROUPS):
            iv = idx_v[s, pl.ds(g * LANES, LANES)]
            cbase.append(lax.shift_left(iv & three, 5))
        for d in range(EMBED):
            wv = plsc.load_gather(w_v, [wbase + d])
            for g in range(GROUPS):
                col = plsc.load_gather(
                    rows, [iota + (g * LANES), cbase[g] + d])
                out[g] = out[g] + col * wv
        @pl.when(s + NBUF < SEQ)
        def _():
            pltpu.async_copy(t2_hbm.at[row_v.at[s + NBUF]],
                             rows_v.at[buf], sems[buf])
        return tuple(out)

    def outer(s2, accs):
        for half in range(NBUF):
            accs = step(s2 * NBUF + half, half, accs)
        return accs

    accs = tuple(jnp.zeros((LANES,), jnp.float32) for _ in range(GROUPS))
    accs = lax.fori_loop(0, SEQ // NBUF, outer, accs)

    bias = plsc.load_gather(
        w_v, [jnp.full((LANES,), SEQ * EMBED, jnp.int32)])
    for g in range(GROUPS):
        out_v[pl.ds(g * LANES, LANES)] = accs[g] + bias
    pltpu.sync_copy(out_v, out_hbm.at[pl.ds(base, BPW)])


@jax.jit
def _run(x1d, wfull, tableT, tail16):
    t2 = _mesh_kernel(
        _tr_body,
        jax.ShapeDtypeStruct((PROWS, PACK * EMBED), jnp.float32),
        [
            pltpu.VMEM((NBUF, EMBED, TB), jnp.float32),
            pltpu.VMEM((NBUF, TOUT, PACK * EMBED), jnp.float32),
            pltpu.SemaphoreType.DMA,
            pltpu.SemaphoreType.DMA,
            pltpu.SemaphoreType.DMA,
            pltpu.SemaphoreType.DMA,
        ],
    )(tableT, tail16)
    return _mesh_kernel(
        _body,
        jax.ShapeDtypeStruct((BATCH,), jnp.float32),
        [
            pltpu.VMEM((SEQ, BPW), jnp.int32),
            pltpu.VMEM((SEQ, BPW), jnp.int32),
            pltpu.VMEM((NBUF, BPW, PACK * EMBED), jnp.float32),
            pltpu.VMEM((WPAD,), jnp.float32),
            pltpu.VMEM((BPW,), jnp.float32),
            pltpu.SemaphoreType.DMA,
            pltpu.SemaphoreType.DMA,
            pltpu.SemaphoreType.DMA,
        ],
    )(x1d, wfull, t2)


def kernel(x, table, W, b):
    wfull = jnp.concatenate(
        [W[:, 0], b, jnp.zeros((LANES - 1,), jnp.float32)])
    x1d = x.reshape(-1)
    tail16 = table[VOCAB - 64:, :].reshape(16, PACK * EMBED)
    return _run(x1d, wfull, table.T, tail16)


# speedup vs baseline: 2.7302x; 2.7302x over previous
import functools

import jax
import jax.numpy as jnp
from jax import lax
from jax.experimental import pallas as pl
from jax.experimental.pallas import tpu as pltpu
from jax.experimental.pallas import tpu_sc as plsc

SEQ = 50
BATCH = 4096
EMBED = 32
LANES = 16
NC = 2
NS = 16
NW = NC * NS
BPW = BATCH // NW
GROUPS = BPW // LANES
NBUF = 2
VOCAB = 1000000
PACK = 4
PROWS = VOCAB // PACK
WPAD = SEQ * EMBED + LANES

TB = 512
TOUT = TB // PACK
NBLK = VOCAB // TB
FULL_ROUNDS = 61


def _mesh_kernel(body, out_shape, scratch):
    mesh = plsc.VectorSubcoreMesh(core_axis_name="c", subcore_axis_name="s")
    return functools.partial(
        pl.kernel,
        out_type=out_shape,
        mesh=mesh,
        compiler_params=pltpu.CompilerParams(
            needs_layout_passes=False, use_tc_tiling_on_sc=True),
        scratch_types=scratch,
    )(body)


CUNROLL = 8


def _tr_block(in_ref, out_ref, iota, n=None):
    if n is None:
        n = out_ref.shape[0]
    ncols = n * 4
    dhi = iota + LANES

    def chunk(c8, _):
        for cc in range(CUNROLL):
            c0 = c8 * CUNROLL + cc
            cw = (jnp.broadcast_to(c0, (LANES,)) + iota) & (ncols - 1)
            row = lax.shift_right_logical(cw, 2)
            col = lax.shift_left(cw & 3, 5)
            vlo = plsc.load_gather(in_ref, [iota, cw])
            vhi = plsc.load_gather(in_ref, [dhi, cw])
            plsc.store_scatter(out_ref, [row, col + iota], vlo)
            plsc.store_scatter(out_ref, [row, col + dhi], vhi)
        return 0

    lax.fori_loop(0, ncols // CUNROLL, chunk, 0)


def _tr_body(tT_hbm, tail_hbm, t2_hbm, in_v, out_v, si0, si1, so0, so1):
    wid = lax.axis_index("s") * NC + lax.axis_index("c")
    iota = lax.iota(jnp.int32, LANES)
    sin = [si0, si1]
    sout = [so0, so1]

    def in_copy(k, buf):
        b = wid + k * NW
        off = pl.multiple_of(b * TB, 128)
        return pltpu.make_async_copy(
            tT_hbm.at[:, pl.ds(off, TB)], in_v.at[buf], sin[buf])

    def out_copy(k, buf):
        b = wid + k * NW
        off = pl.multiple_of(b * TOUT, 8)
        return pltpu.make_async_copy(
            out_v.at[buf], t2_hbm.at[pl.ds(off, TOUT), :], sout[buf])

    in_copy(0, 0).start()
    in_copy(1, 1).start()

    def round_(k, buf):
        in_copy(k, buf).wait()

        @pl.when(k >= NBUF)
        def _():
            out_copy(k - NBUF, buf).wait()
        _tr_block(in_v.at[buf], out_v.at[buf], iota)

        @pl.when(k + NBUF < FULL_ROUNDS)
        def _():
            in_copy(k + NBUF, buf).start()
        out_copy(k, buf).start()

    def outer(k2, carry):
        for half in range(NBUF):
            round_(k2 * NBUF + half, half)
        return carry

    lax.fori_loop(0, FULL_ROUNDS // NBUF, outer, 0)
    round_(FULL_ROUNDS - 1, 0)
    out_copy(FULL_ROUNDS - 2, 1).wait()
    out_copy(FULL_ROUNDS - 1, 0).wait()

    @pl.when(wid == 0)
    def _():
        b = NBLK - 1
        pltpu.sync_copy(tT_hbm.at[:, pl.ds(b * TB, TB)], in_v.at[0])
        _tr_block(in_v.at[0], out_v.at[0], iota)
        pltpu.sync_copy(out_v.at[0], t2_hbm.at[pl.ds(b * TOUT, TOUT), :])

    @pl.when(wid == 1)
    def _():
        pltpu.sync_copy(tail_hbm, out_v.at[1].at[pl.ds(0, 16)])
        pltpu.sync_copy(out_v.at[1].at[pl.ds(0, 16)],
                        t2_hbm.at[pl.ds(PROWS - 16, 16), :])


def _body(x_hbm, w_hbm, t2_hbm, out_hbm, idx_v, row_v, rows_v, w_v, out_v,
          sem0, sem1, semi):
    wid = lax.axis_index("s") * NC + lax.axis_index("c")
    base = wid * BPW

    for s in range(SEQ):
        pltpu.async_copy(x_hbm.at[pl.ds(s * BATCH + base, BPW)],
                         idx_v.at[s], semi)
    pltpu.sync_copy(w_hbm, w_v)
    for s in range(SEQ):
        pltpu.make_async_copy(x_hbm.at[pl.ds(s * BATCH + base, BPW)],
                              idx_v.at[s], semi).wait()

    for s in range(SEQ):
        for g in range(GROUPS):
            iv = idx_v[s, pl.ds(g * LANES, LANES)]
            row_v[s, pl.ds(g * LANES, LANES)] = lax.shift_right_logical(
                iv, 2)

    sems = [sem0, sem1]
    for buf in range(NBUF):
        pltpu.async_copy(t2_hbm.at[row_v.at[buf]], rows_v.at[buf],
                         sems[buf])

    iota = lax.iota(jnp.int32, LANES)
    three = jnp.full((LANES,), 3, jnp.int32)

    def step(s, buf, accs):
        pltpu.make_async_copy(t2_hbm.at[row_v.at[s]], rows_v.at[buf],
                              sems[buf]).wait()
        rows = rows_v.at[buf]
        out = list(accs)
        wbase = jnp.broadcast_to(s * EMBED, (LANES,))
        cbase = []
        for g in range(GROUPS):
            iv = idx_v[s, pl.ds(g * LANES, LANES)]
            cbase.append(lax.shift_left(iv & three, 5))
        for d in range(EMBED):
            wv = plsc.load_gather(w_v, [wbase + d])
            for g in range(GROUPS):
                col = plsc.load_gather(
                    rows, [iota + (g * LANES), cbase[g] + d])
                out[g] = out[g] + col * wv
        @pl.when(s + NBUF < SEQ)
        def _():
            pltpu.async_copy(t2_hbm.at[row_v.at[s + NBUF]],
                             rows_v.at[buf], sems[buf])
        return tuple(out)

    def outer(s2, accs):
        for half in range(NBUF):
            accs = step(s2 * NBUF + half, half, accs)
        return accs

    accs = tuple(jnp.zeros((LANES,), jnp.float32) for _ in range(GROUPS))
    accs = lax.fori_loop(0, SEQ // NBUF, outer, accs)

    bias = plsc.load_gather(
        w_v, [jnp.full((LANES,), SEQ * EMBED, jnp.int32)])
    for g in range(GROUPS):
        out_v[pl.ds(g * LANES, LANES)] = accs[g] + bias
    pltpu.sync_copy(out_v, out_hbm.at[pl.ds(base, BPW)])


@jax.jit
def _run(x1d, wfull, tableT, tail16):
    t2 = _mesh_kernel(
        _tr_body,
        jax.ShapeDtypeStruct((PROWS, PACK * EMBED), jnp.float32),
        [
            pltpu.VMEM((NBUF, EMBED, TB), jnp.float32),
            pltpu.VMEM((NBUF, TOUT, PACK * EMBED), jnp.float32),
            pltpu.SemaphoreType.DMA,
            pltpu.SemaphoreType.DMA,
            pltpu.SemaphoreType.DMA,
            pltpu.SemaphoreType.DMA,
        ],
    )(tableT, tail16)
    return _mesh_kernel(
        _body,
        jax.ShapeDtypeStruct((BATCH,), jnp.float32),
        [
            pltpu.VMEM((SEQ, BPW), jnp.int32),
            pltpu.VMEM((SEQ, BPW), jnp.int32),
            pltpu.VMEM((NBUF, BPW, PACK * EMBED), jnp.float32),
            pltpu.VMEM((WPAD,), jnp.float32),
            pltpu.VMEM((BPW,), jnp.float32),
            pltpu.SemaphoreType.DMA,
            pltpu.SemaphoreType.DMA,
            pltpu.SemaphoreType.DMA,
        ],
    )(x1d, wfull, t2)


def kernel(x, table, W, b):
    wfull = jnp.concatenate(
        [W[:, 0], b, jnp.zeros((LANES - 1,), jnp.float32)])
    x1d = x.reshape(-1)
    tail16 = table[VOCAB - 64:, :].reshape(16, PACK * EMBED)
    return _run(x1d, wfull, table.T, tail16)


# speedup vs baseline: 3.3889x; 1.2413x over previous
import functools

import jax
import jax.numpy as jnp
from jax import lax
from jax.experimental import pallas as pl
from jax.experimental.pallas import tpu as pltpu
from jax.experimental.pallas import tpu_sc as plsc

SEQ = 50
BATCH = 4096
EMBED = 32
LANES = 16
NC = 2
NS = 16
NW = NC * NS
BPW = BATCH // NW
GROUPS = BPW // LANES
NBUF = 2
VOCAB = 1000000
PACK = 4
PROWS = VOCAB // PACK
WPAD = SEQ * EMBED + LANES

TB = 512
TOUT = TB // PACK
NBLK = VOCAB // TB
FULL_ROUNDS = 61


def _mesh_kernel(body, out_shape, scratch):
    mesh = plsc.VectorSubcoreMesh(core_axis_name="c", subcore_axis_name="s")
    return functools.partial(
        pl.kernel,
        out_type=out_shape,
        mesh=mesh,
        compiler_params=pltpu.CompilerParams(
            needs_layout_passes=False, use_tc_tiling_on_sc=True),
        scratch_types=scratch,
    )(body)


CUNROLL = 8


def _tr_block(in_ref, out_ref, iota, n=None):
    if n is None:
        n = out_ref.shape[0]
    ncols = n * 4
    dhi = iota + LANES

    def chunk(c8, _):
        for cc in range(CUNROLL):
            c0 = c8 * CUNROLL + cc
            cw = (jnp.broadcast_to(c0, (LANES,)) + iota) & (ncols - 1)
            row = lax.shift_right_logical(cw, 2)
            col = lax.shift_left(cw & 3, 5)
            vlo = plsc.load_gather(in_ref, [iota, cw])
            vhi = plsc.load_gather(in_ref, [dhi, cw])
            plsc.store_scatter(out_ref, [row, col + iota], vlo)
            plsc.store_scatter(out_ref, [row, col + dhi], vhi)
        return 0

    lax.fori_loop(0, ncols // CUNROLL, chunk, 0)


def _tr_body(tT_hbm, tail_hbm, t2_hbm, in_v, out_v, si0, si1, so0, so1):
    wid = lax.axis_index("s") * NC + lax.axis_index("c")
    iota = lax.iota(jnp.int32, LANES)
    sin = [si0, si1]
    sout = [so0, so1]

    def in_copy(k, buf):
        b = wid + k * NW
        off = pl.multiple_of(b * TB, 128)
        return pltpu.make_async_copy(
            tT_hbm.at[:, pl.ds(off, TB)], in_v.at[buf], sin[buf])

    def out_copy(k, buf):
        b = wid + k * NW
        off = pl.multiple_of(b * TOUT, 8)
        return pltpu.make_async_copy(
            out_v.at[buf], t2_hbm.at[pl.ds(off, TOUT), :], sout[buf])

    in_copy(0, 0).start()
    in_copy(1, 1).start()

    def round_(k, buf):
        in_copy(k, buf).wait()

        @pl.when(k >= NBUF)
        def _():
            out_copy(k - NBUF, buf).wait()
        _tr_block(in_v.at[buf], out_v.at[buf], iota)

        @pl.when(k + NBUF < FULL_ROUNDS)
        def _():
            in_copy(k + NBUF, buf).start()
        out_copy(k, buf).start()

    def outer(k2, carry):
        for half in range(NBUF):
            round_(k2 * NBUF + half, half)
        return carry

    lax.fori_loop(0, FULL_ROUNDS // NBUF, outer, 0)
    round_(FULL_ROUNDS - 1, 0)
    out_copy(FULL_ROUNDS - 2, 1).wait()
    out_copy(FULL_ROUNDS - 1, 0).wait()

    @pl.when(wid == 0)
    def _():
        b = NBLK - 1
        pltpu.sync_copy(tT_hbm.at[:, pl.ds(b * TB, TB)], in_v.at[0])
        _tr_block(in_v.at[0], out_v.at[0], iota)
        pltpu.sync_copy(out_v.at[0], t2_hbm.at[pl.ds(b * TOUT, TOUT), :])

    @pl.when(wid == 1)
    def _():
        pltpu.sync_copy(tail_hbm, out_v.at[1].at[pl.ds(0, 16)])
        pltpu.sync_copy(out_v.at[1].at[pl.ds(0, 16)],
                        t2_hbm.at[pl.ds(PROWS - 16, 16), :])


def _body(x_hbm, w_hbm, t2_hbm, out_hbm, idx_v, row_v, rows_v, w_v, out_v,
          sem0, sem1, semi):
    wid = lax.axis_index("s") * NC + lax.axis_index("c")
    base = wid * BPW

    for s in range(SEQ):
        pltpu.async_copy(x_hbm.at[pl.ds(s * BATCH + base, BPW)],
                         idx_v.at[s], semi)
    pltpu.sync_copy(w_hbm, w_v)
    for s in range(SEQ):
        pltpu.make_async_copy(x_hbm.at[pl.ds(s * BATCH + base, BPW)],
                              idx_v.at[s], semi).wait()

    for s in range(SEQ):
        for g in range(GROUPS):
            iv = idx_v[s, pl.ds(g * LANES, LANES)]
            row_v[s, pl.ds(g * LANES, LANES)] = lax.shift_right_logical(
                iv, 2)

    sems = [sem0, sem1]
    for buf in range(NBUF):
        pltpu.async_copy(t2_hbm.at[row_v.at[buf]], rows_v.at[buf],
                         sems[buf])

    iota = lax.iota(jnp.int32, LANES)
    three = jnp.full((LANES,), 3, jnp.int32)

    def step(s, buf, accs):
        pltpu.make_async_copy(t2_hbm.at[row_v.at[s]], rows_v.at[buf],
                              sems[buf]).wait()
        rows = rows_v.at[buf]
        out = list(accs)
        wbase = jnp.broadcast_to(s * EMBED, (LANES,))
        wlo = plsc.load_gather(w_v, [wbase + iota])
        whi = plsc.load_gather(w_v, [wbase + LANES + iota])
        cbase = []
        for g in range(GROUPS):
            iv = idx_v[s, pl.ds(g * LANES, LANES)]
            cbase.append(lax.shift_left(iv & three, 5))
        for d0 in range(EMBED):
            dvec = (jnp.broadcast_to(d0, (LANES,)) + iota) & 31
            d16 = dvec & 15
            wr = jnp.where(dvec < LANES,
                           wlo[d16], whi[d16])
            for g in range(GROUPS):
                col = plsc.load_gather(
                    rows, [iota + (g * LANES), cbase[g] + dvec])
                out[g] = out[g] + col * wr
        @pl.when(s + NBUF < SEQ)
        def _():
            pltpu.async_copy(t2_hbm.at[row_v.at[s + NBUF]],
                             rows_v.at[buf], sems[buf])
        return tuple(out)

    def outer(s2, accs):
        for half in range(NBUF):
            accs = step(s2 * NBUF + half, half, accs)
        return accs

    accs = tuple(jnp.zeros((LANES,), jnp.float32) for _ in range(GROUPS))
    accs = lax.fori_loop(0, SEQ // NBUF, outer, accs)

    bias = plsc.load_gather(
        w_v, [jnp.full((LANES,), SEQ * EMBED, jnp.int32)])
    for g in range(GROUPS):
        out_v[pl.ds(g * LANES, LANES)] = accs[g] + bias
    pltpu.sync_copy(out_v, out_hbm.at[pl.ds(base, BPW)])


@jax.jit
def _run(x1d, wfull, tableT, tail16):
    t2 = _mesh_kernel(
        _tr_body,
        jax.ShapeDtypeStruct((PROWS, PACK * EMBED), jnp.float32),
        [
            pltpu.VMEM((NBUF, EMBED, TB), jnp.float32),
            pltpu.VMEM((NBUF, TOUT, PACK * EMBED), jnp.float32),
            pltpu.SemaphoreType.DMA,
            pltpu.SemaphoreType.DMA,
            pltpu.SemaphoreType.DMA,
            pltpu.SemaphoreType.DMA,
        ],
    )(tableT, tail16)
    return _mesh_kernel(
        _body,
        jax.ShapeDtypeStruct((BATCH,), jnp.float32),
        [
            pltpu.VMEM((SEQ, BPW), jnp.int32),
            pltpu.VMEM((SEQ, BPW), jnp.int32),
            pltpu.VMEM((NBUF, BPW, PACK * EMBED), jnp.float32),
            pltpu.VMEM((WPAD,), jnp.float32),
            pltpu.VMEM((BPW,), jnp.float32),
            pltpu.SemaphoreType.DMA,
            pltpu.SemaphoreType.DMA,
            pltpu.SemaphoreType.DMA,
        ],
    )(x1d, wfull, t2)


def kernel(x, table, W, b):
    wfull = jnp.concatenate(
        [W[:, 0], b, jnp.zeros((LANES - 1,), jnp.float32)])
    x1d = x.reshape(-1)
    tail16 = table[VOCAB - 64:, :].reshape(16, PACK * EMBED)
    return _run(x1d, wfull, table.T, tail16)
